# baseline (device time: 134310 ns/iter reference)
import jax
import jax.numpy as jnp
from jax import lax
from jax.experimental import pallas as pl
from jax.experimental.pallas import tpu as pltpu

N_X = 2
K = 8


def kernel(x):
    m, n = x.shape
    half = m // 2
    c = half // K
    n_chunks = m // c

    def body(x_hbm, out_hbm, own_bf16, miss_bf16, stag, stag_sem,
             x_send, x_recv, y_send, y_recv, own_sem, miss_sem):
        my_x = lax.axis_index("x")
        my_y = lax.axis_index("y")
        my_z = lax.axis_index("z")
        x_peer = (1 - my_x, my_y, my_z)
        y_peer = (my_x, 1 - my_y, my_z)

        miss = (1 - my_x) * m
        mine = my_y * half

        barrier_sem = pltpu.get_barrier_semaphore()
        for nbr in [x_peer, y_peer]:
            pl.semaphore_signal(
                barrier_sem, inc=1, device_id=nbr,
                device_id_type=pl.DeviceIdType.MESH,
            )
        pl.semaphore_wait(barrier_sem, 2)

        other = (1 - my_y) * half

        def stage_start(off, slot):
            cp = pltpu.make_async_copy(
                x_hbm.at[pl.ds(off, c), :], stag.at[slot], stag_sem.at[slot],
            )
            cp.start()
            return cp

        x_rdmas = []
        cps = {0: stage_start(mine, 0)}
        for k in range(K):
            slot = k % 2
            if k + 1 < K:
                cps[(k + 1) % 2] = stage_start(mine + (k + 1) * c, (k + 1) % 2)
            cps[slot].wait()
            own_bf16[pl.ds(mine + k * c, c), :] = (
                stag[slot, :, :].astype(jnp.bfloat16))
            rdma = pltpu.make_async_remote_copy(
                src_ref=own_bf16.at[pl.ds(mine + k * c, c), :],
                dst_ref=miss_bf16.at[pl.ds(mine + k * c, c), :],
                send_sem=x_send.at[k],
                recv_sem=x_recv.at[k],
                device_id=x_peer,
                device_id_type=pl.DeviceIdType.MESH,
            )
            rdma.start()
            x_rdmas.append(rdma)

        own_a = pltpu.make_async_copy(
            own_bf16.at[pl.ds(mine, half), :],
            out_hbm.at[pl.ds(my_x * m + mine, half), :],
            own_sem.at[0],
        )
        own_a.start()

        y_rdmas = []
        cps = {0: stage_start(other, 0)}
        for k in range(K):
            x_rdmas[k].wait_recv()
            rdma = pltpu.make_async_remote_copy(
                src_ref=miss_bf16.at[pl.ds(mine + k * c, c), :],
                dst_ref=miss_bf16.at[pl.ds(mine + k * c, c), :],
                send_sem=y_send.at[k],
                recv_sem=y_recv.at[k],
                device_id=y_peer,
                device_id_type=pl.DeviceIdType.MESH,
            )
            rdma.start()
            y_rdmas.append(rdma)
            slot = k % 2
            if k + 1 < K:
                cps[(k + 1) % 2] = stage_start(other + (k + 1) * c, (k + 1) % 2)
            cps[slot].wait()
            own_bf16[pl.ds(other + k * c, c), :] = (
                stag[slot, :, :].astype(jnp.bfloat16))

        own_b = pltpu.make_async_copy(
            own_bf16.at[pl.ds(other, half), :],
            out_hbm.at[pl.ds(my_x * m + other, half), :],
            own_sem.at[1],
        )
        own_b.start()

        miss_a = pltpu.make_async_copy(
            miss_bf16.at[pl.ds(mine, half), :],
            out_hbm.at[pl.ds(miss + mine, half), :],
            miss_sem.at[0],
        )
        miss_a.start()

        for k in range(K):
            y_rdmas[k].wait_recv()
        miss_b = pltpu.make_async_copy(
            miss_bf16.at[pl.ds(other, half), :],
            out_hbm.at[pl.ds(miss + other, half), :],
            miss_sem.at[1],
        )
        miss_b.start()
        for k in range(K):
            x_rdmas[k].wait_send()
            y_rdmas[k].wait_send()
        own_a.wait()
        own_b.wait()
        miss_a.wait()
        miss_b.wait()

    return pl.pallas_call(
        body,
        out_shape=jax.ShapeDtypeStruct((N_X * m, n), jnp.bfloat16),
        in_specs=[pl.BlockSpec(memory_space=pl.ANY)],
        out_specs=pl.BlockSpec(memory_space=pl.ANY),
        scratch_shapes=[
            pltpu.VMEM((m, n), jnp.bfloat16),
            pltpu.VMEM((m, n), jnp.bfloat16),
            pltpu.VMEM((2, c, n), jnp.float32),
            pltpu.SemaphoreType.DMA((2,)),
            pltpu.SemaphoreType.DMA((K,)),
            pltpu.SemaphoreType.DMA((K,)),
            pltpu.SemaphoreType.DMA((K,)),
            pltpu.SemaphoreType.DMA((K,)),
            pltpu.SemaphoreType.DMA((2,)),
            pltpu.SemaphoreType.DMA((2,)),
        ],
        compiler_params=pltpu.CompilerParams(
            collective_id=0,
            vmem_limit_bytes=48 * 1024 * 1024,
        ),
    )(x)


# device time: 111708 ns/iter; 1.2023x vs baseline; 1.2023x over previous
import jax
import jax.numpy as jnp
from jax import lax
from jax.experimental import pallas as pl
from jax.experimental.pallas import tpu as pltpu

N_X = 2
CH = 512
NQ = 4


def kernel(x):
    m, n = x.shape
    quarter = m // 4
    n_conv = m // CH

    def body(x_hbm, out_hbm, own_bf16, miss_bf16, stag, stag_sem,
             x_send, x_recv, yf_send, yf_recv, zf_send, zf_recv,
             yh_send, yh_recv, zh_send, zh_recv, own_sem, miss_sem):
        my_x = lax.axis_index("x")
        my_y = lax.axis_index("y")
        my_z = lax.axis_index("z")
        x_peer = (1 - my_x, my_y, my_z)
        y_peer = (my_x, 1 - my_y, my_z)
        z_peer = (my_x, my_y, 1 - my_z)

        miss = (1 - my_x) * m
        q_me = 2 * my_y + my_z
        q_yn = 2 * (1 - my_y) + my_z
        q_zn = 2 * my_y + (1 - my_z)
        q_d = 2 * (1 - my_y) + (1 - my_z)
        qoff = q_me * quarter

        barrier_sem = pltpu.get_barrier_semaphore()
        for nbr in [x_peer, y_peer, z_peer]:
            pl.semaphore_signal(
                barrier_sem, inc=1, device_id=nbr,
                device_id_type=pl.DeviceIdType.MESH,
            )
        pl.semaphore_wait(barrier_sem, 3)

        def conv_off(i):
            return (qoff + i * CH) % m

        def stage_start(i):
            cp = pltpu.make_async_copy(
                x_hbm.at[pl.ds(conv_off(i), CH), :],
                stag.at[i % 2], stag_sem.at[i % 2],
            )
            cp.start()
            return cp

        def do_convert(i):
            if i + 1 < n_conv:
                stage_start(i + 1)
            pltpu.make_async_copy(
                x_hbm.at[pl.ds(conv_off(i), CH), :],
                stag.at[i % 2], stag_sem.at[i % 2],
            ).wait()
            own_bf16[pl.ds(conv_off(i), CH), :] = (
                stag[i % 2, :, :].astype(jnp.bfloat16))

        def remote(src_ref, dst_ref, ssem, rsem, peer):
            r = pltpu.make_async_remote_copy(
                src_ref=src_ref, dst_ref=dst_ref, send_sem=ssem,
                recv_sem=rsem, device_id=peer,
                device_id_type=pl.DeviceIdType.MESH,
            )
            r.start()
            return r

        stage_start(0)
        x_rdmas = []
        for i in range(NQ):
            do_convert(i)
            x_rdmas.append(remote(
                own_bf16.at[pl.ds(qoff + i * CH, CH), :],
                miss_bf16.at[pl.ds(qoff + i * CH, CH), :],
                x_send.at[i], x_recv.at[i], x_peer,
            ))

        yf_rdmas, zf_rdmas = [], []
        for i in range(NQ):
            x_rdmas[i].wait_recv()
            src = miss_bf16.at[pl.ds(qoff + i * CH, CH), :]
            yf_rdmas.append(remote(
                src, miss_bf16.at[pl.ds(qoff + i * CH, CH), :],
                yf_send.at[i], yf_recv.at[i], y_peer,
            ))
            zf_rdmas.append(remote(
                src, miss_bf16.at[pl.ds(qoff + i * CH, CH), :],
                zf_send.at[i], zf_recv.at[i], z_peer,
            ))
            do_convert(NQ + i)

        miss_dmas = []
        cp = pltpu.make_async_copy(
            miss_bf16.at[pl.ds(qoff, quarter), :],
            out_hbm.at[pl.ds(miss + qoff, quarter), :], miss_sem.at[0],
        )
        cp.start()
        miss_dmas.append(cp)

        yh_rdmas, zh_rdmas = [], []
        for i in range(NQ):
            zf_rdmas[i].wait_recv()
            if i >= 2:
                yh_rdmas.append(remote(
                    miss_bf16.at[pl.ds(q_zn * quarter + i * CH, CH), :],
                    miss_bf16.at[pl.ds(q_zn * quarter + i * CH, CH), :],
                    yh_send.at[i - 2], yh_recv.at[i - 2], y_peer,
                ))
            yf_rdmas[i].wait_recv()
            if i < 2:
                zh_rdmas.append(remote(
                    miss_bf16.at[pl.ds(q_yn * quarter + i * CH, CH), :],
                    miss_bf16.at[pl.ds(q_yn * quarter + i * CH, CH), :],
                    zh_send.at[i], zh_recv.at[i], z_peer,
                ))
            do_convert(2 * NQ + 2 * i)
            do_convert(2 * NQ + 2 * i + 1)

        own_cp = pltpu.make_async_copy(
            own_bf16, out_hbm.at[pl.ds(my_x * m, m), :], own_sem,
        )
        own_cp.start()
        for si, q in ((1, q_zn), (2, q_yn)):
            cp = pltpu.make_async_copy(
                miss_bf16.at[pl.ds(q * quarter, quarter), :],
                out_hbm.at[pl.ds(miss + q * quarter, quarter), :],
                miss_sem.at[si],
            )
            cp.start()
            miss_dmas.append(cp)

        for i in range(2):
            yh_rdmas[i].wait_recv()
            zh_rdmas[i].wait_recv()
        cp = pltpu.make_async_copy(
            miss_bf16.at[pl.ds(q_d * quarter, quarter), :],
            out_hbm.at[pl.ds(miss + q_d * quarter, quarter), :],
            miss_sem.at[3],
        )
        cp.start()
        miss_dmas.append(cp)

        for r in x_rdmas + yf_rdmas + zf_rdmas + yh_rdmas + zh_rdmas:
            r.wait_send()
        own_cp.wait()
        for cp in miss_dmas:
            cp.wait()

    return pl.pallas_call(
        body,
        out_shape=jax.ShapeDtypeStruct((N_X * m, n), jnp.bfloat16),
        in_specs=[pl.BlockSpec(memory_space=pl.ANY)],
        out_specs=pl.BlockSpec(memory_space=pl.ANY),
        scratch_shapes=[
            pltpu.VMEM((m, n), jnp.bfloat16),
            pltpu.VMEM((m, n), jnp.bfloat16),
            pltpu.VMEM((2, CH, n), jnp.float32),
            pltpu.SemaphoreType.DMA((2,)),
            pltpu.SemaphoreType.DMA((NQ,)),
            pltpu.SemaphoreType.DMA((NQ,)),
            pltpu.SemaphoreType.DMA((NQ,)),
            pltpu.SemaphoreType.DMA((NQ,)),
            pltpu.SemaphoreType.DMA((NQ,)),
            pltpu.SemaphoreType.DMA((NQ,)),
            pltpu.SemaphoreType.DMA((2,)),
            pltpu.SemaphoreType.DMA((2,)),
            pltpu.SemaphoreType.DMA((2,)),
            pltpu.SemaphoreType.DMA((2,)),
            pltpu.SemaphoreType.DMA,
            pltpu.SemaphoreType.DMA((4,)),
        ],
        compiler_params=pltpu.CompilerParams(
            collective_id=0,
            vmem_limit_bytes=48 * 1024 * 1024,
        ),
    )(x)


# device time: 104827 ns/iter; 1.2813x vs baseline; 1.0656x over previous
import jax
import jax.numpy as jnp
from jax import lax
from jax.experimental import pallas as pl
from jax.experimental.pallas import tpu as pltpu

N_X = 2
CH = 512
NQ = 4
D_X = 688
D_Y = 688
D_Z = 672


def kernel(x):
    m, n = x.shape
    quarter = m // 4
    n_conv = m // CH

    def body(x_hbm, out_hbm, own_bf16, miss_bf16, stag, stag_sem,
             x_send, x_recv, x2_send, x2_recv,
             yf_send, yf_recv, zf_send, zf_recv,
             yh_send, yh_recv, zh_send, zh_recv, own_sem, miss_sem):
        my_x = lax.axis_index("x")
        my_y = lax.axis_index("y")
        my_z = lax.axis_index("z")
        x_peer = (1 - my_x, my_y, my_z)
        y_peer = (my_x, 1 - my_y, my_z)
        z_peer = (my_x, my_y, 1 - my_z)

        miss = (1 - my_x) * m
        q_me = 2 * my_y + my_z
        q_yn = 2 * (1 - my_y) + my_z
        q_zn = 2 * my_y + (1 - my_z)
        q_d = 2 * (1 - my_y) + (1 - my_z)
        qoff = q_me * quarter
        doff = q_d * quarter

        barrier_sem = pltpu.get_barrier_semaphore()
        for nbr in [x_peer, y_peer, z_peer]:
            pl.semaphore_signal(
                barrier_sem, inc=1, device_id=nbr,
                device_id_type=pl.DeviceIdType.MESH,
            )
        pl.semaphore_wait(barrier_sem, 3)

        qseq = [q_me, q_d, q_yn, q_zn]

        def conv_off(i):
            return qseq[i // NQ] * quarter + (i % NQ) * CH

        def stage_start(i):
            cp = pltpu.make_async_copy(
                x_hbm.at[pl.ds(conv_off(i), CH), :],
                stag.at[i % 2], stag_sem.at[i % 2],
            )
            cp.start()
            return cp

        def do_convert(i):
            if i + 1 < n_conv:
                stage_start(i + 1)
            pltpu.make_async_copy(
                x_hbm.at[pl.ds(conv_off(i), CH), :],
                stag.at[i % 2], stag_sem.at[i % 2],
            ).wait()
            own_bf16[pl.ds(conv_off(i), CH), :] = (
                stag[i % 2, :, :].astype(jnp.bfloat16))

        def remote(src_ref, dst_ref, ssem, rsem, peer):
            r = pltpu.make_async_remote_copy(
                src_ref=src_ref, dst_ref=dst_ref, send_sem=ssem,
                recv_sem=rsem, device_id=peer,
                device_id_type=pl.DeviceIdType.MESH,
            )
            r.start()
            return r

        stage_start(0)
        x_rdmas = []
        for i in range(NQ):
            do_convert(i)
            x_rdmas.append(remote(
                own_bf16.at[pl.ds(qoff + i * CH, CH), :],
                miss_bf16.at[pl.ds(qoff + i * CH, CH), :],
                x_send.at[i], x_recv.at[i], x_peer,
            ))
        do_convert(NQ)
        do_convert(NQ + 1)
        x2_rdma = remote(
            own_bf16.at[pl.ds(doff, D_X), :],
            miss_bf16.at[pl.ds(doff, D_X), :],
            x2_send, x2_recv, x_peer,
        )

        yf_rdmas, zf_rdmas = [], []
        for i in range(NQ):
            x_rdmas[i].wait_recv()
            src = miss_bf16.at[pl.ds(qoff + i * CH, CH), :]
            yf_rdmas.append(remote(
                src, miss_bf16.at[pl.ds(qoff + i * CH, CH), :],
                yf_send.at[i], yf_recv.at[i], y_peer,
            ))
            zf_rdmas.append(remote(
                src, miss_bf16.at[pl.ds(qoff + i * CH, CH), :],
                zf_send.at[i], zf_recv.at[i], z_peer,
            ))
            do_convert(NQ + 2 + i)

        miss_dmas = []
        cp = pltpu.make_async_copy(
            miss_bf16.at[pl.ds(qoff, quarter), :],
            out_hbm.at[pl.ds(miss + qoff, quarter), :], miss_sem.at[0],
        )
        cp.start()
        miss_dmas.append(cp)

        yh_rdma = zh_rdma = None
        ci = NQ + 2 + NQ
        for i in range(NQ):
            zf_rdmas[i].wait_recv()
            if i == 2:
                yh_rdma = remote(
                    miss_bf16.at[pl.ds(q_zn * quarter + D_X, D_Y), :],
                    miss_bf16.at[pl.ds(q_zn * quarter + D_X, D_Y), :],
                    yh_send, yh_recv, y_peer,
                )
            yf_rdmas[i].wait_recv()
            if i == 3:
                zh_rdma = remote(
                    miss_bf16.at[pl.ds(q_yn * quarter + D_X + D_Y, D_Z), :],
                    miss_bf16.at[pl.ds(q_yn * quarter + D_X + D_Y, D_Z), :],
                    zh_send, zh_recv, z_peer,
                )
            while ci < n_conv and ci < NQ + 2 + NQ + 2 * (i + 1):
                do_convert(ci)
                ci += 1

        own_cp = pltpu.make_async_copy(
            own_bf16, out_hbm.at[pl.ds(my_x * m, m), :], own_sem,
        )
        own_cp.start()
        for si, q in ((1, q_zn), (2, q_yn)):
            cp = pltpu.make_async_copy(
                miss_bf16.at[pl.ds(q * quarter, quarter), :],
                out_hbm.at[pl.ds(miss + q * quarter, quarter), :],
                miss_sem.at[si],
            )
            cp.start()
            miss_dmas.append(cp)

        x2_rdma.wait_recv()
        yh_rdma.wait_recv()
        zh_rdma.wait_recv()
        cp = pltpu.make_async_copy(
            miss_bf16.at[pl.ds(doff, quarter), :],
            out_hbm.at[pl.ds(miss + doff, quarter), :],
            miss_sem.at[3],
        )
        cp.start()
        miss_dmas.append(cp)

        for r in x_rdmas + yf_rdmas + zf_rdmas + [x2_rdma, yh_rdma, zh_rdma]:
            r.wait_send()
        own_cp.wait()
        for cp in miss_dmas:
            cp.wait()

    return pl.pallas_call(
        body,
        out_shape=jax.ShapeDtypeStruct((N_X * m, n), jnp.bfloat16),
        in_specs=[pl.BlockSpec(memory_space=pl.ANY)],
        out_specs=pl.BlockSpec(memory_space=pl.ANY),
        scratch_shapes=[
            pltpu.VMEM((m, n), jnp.bfloat16),
            pltpu.VMEM((m, n), jnp.bfloat16),
            pltpu.VMEM((2, CH, n), jnp.float32),
            pltpu.SemaphoreType.DMA((2,)),
            pltpu.SemaphoreType.DMA((NQ,)),
            pltpu.SemaphoreType.DMA((NQ,)),
            pltpu.SemaphoreType.DMA,
            pltpu.SemaphoreType.DMA,
            pltpu.SemaphoreType.DMA((NQ,)),
            pltpu.SemaphoreType.DMA((NQ,)),
            pltpu.SemaphoreType.DMA((NQ,)),
            pltpu.SemaphoreType.DMA((NQ,)),
            pltpu.SemaphoreType.DMA,
            pltpu.SemaphoreType.DMA,
            pltpu.SemaphoreType.DMA,
            pltpu.SemaphoreType.DMA,
            pltpu.SemaphoreType.DMA,
            pltpu.SemaphoreType.DMA((4,)),
        ],
        compiler_params=pltpu.CompilerParams(
            collective_id=0,
            vmem_limit_bytes=48 * 1024 * 1024,
        ),
    )(x)
